# Initial kernel scaffold; baseline (speedup 1.0000x reference)
import jax
import jax.numpy as jnp
from jax.experimental import pallas as pl


def kernel(x, edge_index, W1, b1, W2, b2, Wm1, bm1, Wm2, bm2):
    raise NotImplementedError("write your pallas kernel here")


# speedup vs baseline: 2.4979x; 2.4979x over previous
import functools

import jax
import jax.numpy as jnp
from jax.experimental import pallas as pl

N = 50000
E = 800000
H = 64

EDGE_BLK = 8192


def _edge_mlp_body(ar_ref, br_ref, w_ref, o_ref):
    z = jnp.maximum(ar_ref[...] + br_ref[...], 0.0)
    o_ref[...] = (z @ w_ref[...]).reshape(EDGE_BLK // 128, 128)


def _edge_mlp(ar, br, w):
    grid = (pl.cdiv(E, EDGE_BLK),)
    out = pl.pallas_call(
        _edge_mlp_body,
        grid=grid,
        in_specs=[
            pl.BlockSpec((EDGE_BLK, H), lambda i: (i, 0)),
            pl.BlockSpec((EDGE_BLK, H), lambda i: (i, 0)),
            pl.BlockSpec((H, 1), lambda i: (0, 0)),
        ],
        out_specs=pl.BlockSpec((EDGE_BLK // 128, 128), lambda i: (i, 0)),
        out_shape=jax.ShapeDtypeStruct((E // 128, 128), jnp.float32),
    )(ar, br, w)
    return out.reshape(-1)


def kernel(x, edge_index, W1, b1, W2, b2, Wm1, bm1, Wm2, bm2):
    src = edge_index[0].astype(jnp.int32)
    dst = edge_index[1].astype(jnp.int32)

    deg = jnp.ones((N,), jnp.float32).at[dst].add(1.0)
    dinv = jax.lax.rsqrt(deg)

    def layer(h, W, b):
        xs = dinv[:, None] * (h @ W)
        acc = jnp.zeros((N, H), jnp.float32).at[dst].add(xs[src])
        return jax.nn.relu(dinv[:, None] * (acc + xs) + b)

    h = layer(x, W1, b1)
    h = layer(h, W2, b2)

    A = h @ Wm1[:H] + bm1
    B = h @ Wm1[H:]
    out = _edge_mlp(A[src], B[dst], Wm2)
    return out + bm2[0]


# speedup vs baseline: 3.1626x; 1.2661x over previous
import functools

import jax
import jax.numpy as jnp
from jax import lax
from jax.experimental import pallas as pl
from jax.experimental.pallas import tpu as pltpu
from jax.experimental.pallas import tpu_sc as plsc

N = 50000
E = 800000
H = 64

NC = 2
NS = 16
NW = NC * NS

EPW = 25600
E_PAD = NW * EPW
CH = 512
NCHUNK = EPW // CH

_sc_mesh = plsc.VectorSubcoreMesh(core_axis_name="c", subcore_axis_name="s")


def _scorer_body(a_hbm, b_hbm, row_hbm, col_hbm, w_hbm, out_hbm,
                 idxa, idxb, rowsa, rowsb, tbuf, outv, wv, sema, semb):
    c = lax.axis_index("c")
    s = lax.axis_index("s")
    wid = s * NC + c
    base0 = wid * EPW
    pltpu.sync_copy(w_hbm, wv)
    iota16 = lax.iota(jnp.int32, 16)

    def chunk(t, carry):
        base = base0 + t * CH
        pltpu.sync_copy(row_hbm.at[pl.ds(base, CH)], idxa)
        pltpu.sync_copy(col_hbm.at[pl.ds(base, CH)], idxb)
        da = pltpu.async_copy(a_hbm.at[idxa], rowsa, sema)
        db = pltpu.async_copy(b_hbm.at[idxb], rowsb, semb)
        da.wait()
        db.wait()

        wvs = [wv[pl.ds(k * 16, 16)] for k in range(4)]

        def group(gi, carry2):
            e0 = gi * 16
            for e in range(16):
                acc = None
                for k in range(4):
                    av = rowsa[e0 + e, pl.ds(k * 16, 16)]
                    bv = rowsb[e0 + e, pl.ds(k * 16, 16)]
                    z = jnp.maximum(av + bv, 0.0) * wvs[k]
                    acc = z if acc is None else acc + z
                plsc.store_scatter(tbuf, [iota16 * 16 + e], acc)
            tot = tbuf[pl.ds(0, 16)]
            for l in range(1, 16):
                tot = tot + tbuf[pl.ds(l * 16, 16)]
            outv[pl.ds(e0, 16)] = tot
            return carry2

        lax.fori_loop(0, CH // 16, group, 0)
        pltpu.sync_copy(outv, out_hbm.at[pl.ds(base, CH)])
        return carry

    lax.fori_loop(0, NCHUNK, chunk, 0)


_scorer = functools.partial(
    pl.kernel,
    out_type=jax.ShapeDtypeStruct((E_PAD,), jnp.float32),
    mesh=_sc_mesh,
    compiler_params=pltpu.CompilerParams(
        needs_layout_passes=False, use_tc_tiling_on_sc=False
    ),
    scratch_types=[
        pltpu.VMEM((CH,), jnp.int32),
        pltpu.VMEM((CH,), jnp.int32),
        pltpu.VMEM((CH, H), jnp.float32),
        pltpu.VMEM((CH, H), jnp.float32),
        pltpu.VMEM((256,), jnp.float32),
        pltpu.VMEM((CH,), jnp.float32),
        pltpu.VMEM((H,), jnp.float32),
        pltpu.SemaphoreType.DMA,
        pltpu.SemaphoreType.DMA,
    ],
)(_scorer_body)


EDGE_BLK = 8192


def kernel(x, edge_index, W1, b1, W2, b2, Wm1, bm1, Wm2, bm2):
    src = edge_index[0].astype(jnp.int32)
    dst = edge_index[1].astype(jnp.int32)

    deg = jnp.ones((N,), jnp.float32).at[dst].add(1.0)
    dinv = lax.rsqrt(deg)

    def layer(h, W, b):
        xs = dinv[:, None] * (h @ W)
        acc = jnp.zeros((N, H), jnp.float32).at[dst].add(xs[src])
        return jax.nn.relu(dinv[:, None] * (acc + xs) + b)

    h = layer(x, W1, b1)
    h = layer(h, W2, b2)

    A = h @ Wm1[:H] + bm1
    B = h @ Wm1[H:]

    row_pad = jnp.pad(src, (0, E_PAD - E))
    col_pad = jnp.pad(dst, (0, E_PAD - E))
    out = _scorer(A, B, row_pad, col_pad, Wm2[:, 0])
    return out[:E] + bm2[0]


# speedup vs baseline: 5.4679x; 1.7289x over previous
import functools

import jax
import jax.numpy as jnp
from jax import lax
from jax.experimental import pallas as pl
from jax.experimental.pallas import tpu as pltpu
from jax.experimental.pallas import tpu_sc as plsc

N = 50000
E = 800000
H = 64

NC = 2
NS = 16
NW = NC * NS

EPW = 25600
E_PAD = NW * EPW
CH = 512
NCHUNK = EPW // CH

_sc_mesh = plsc.VectorSubcoreMesh(core_axis_name="c", subcore_axis_name="s")


def _scorer_body(a_hbm, b_hbm, row_hbm, col_hbm, w_hbm, out_hbm,
                 idxa, idxb, rowsa, rowsb, tbuf, outv, wv, sema, semb):
    c = lax.axis_index("c")
    s = lax.axis_index("s")
    wid = s * NC + c
    base0 = wid * EPW
    pltpu.sync_copy(w_hbm, wv)
    iota16 = lax.iota(jnp.int32, 16)

    def chunk(t, carry):
        base = base0 + t * CH
        pltpu.sync_copy(row_hbm.at[pl.ds(base, CH)], idxa)
        pltpu.sync_copy(col_hbm.at[pl.ds(base, CH)], idxb)
        da = pltpu.async_copy(a_hbm.at[idxa], rowsa, sema)
        db = pltpu.async_copy(b_hbm.at[idxb], rowsb, semb)
        da.wait()
        db.wait()

        wvs = [wv[pl.ds(k * 16, 16)] for k in range(4)]

        def group(gi, carry2):
            e0 = gi * 16
            for e in range(16):
                acc = None
                for k in range(4):
                    av = rowsa[e0 + e, pl.ds(k * 16, 16)]
                    bv = rowsb[e0 + e, pl.ds(k * 16, 16)]
                    z = jnp.maximum(av + bv, 0.0) * wvs[k]
                    acc = z if acc is None else acc + z
                plsc.store_scatter(tbuf, [iota16 * 16 + e], acc)
            tot = tbuf[pl.ds(0, 16)]
            for l in range(1, 16):
                tot = tot + tbuf[pl.ds(l * 16, 16)]
            outv[pl.ds(e0, 16)] = tot
            return carry2

        lax.fori_loop(0, CH // 16, group, 0)
        pltpu.sync_copy(outv, out_hbm.at[pl.ds(base, CH)])
        return carry

    lax.fori_loop(0, NCHUNK, chunk, 0)


_scorer = functools.partial(
    pl.kernel,
    out_type=jax.ShapeDtypeStruct((E_PAD,), jnp.float32),
    mesh=_sc_mesh,
    compiler_params=pltpu.CompilerParams(
        needs_layout_passes=False, use_tc_tiling_on_sc=False
    ),
    scratch_types=[
        pltpu.VMEM((CH,), jnp.int32),
        pltpu.VMEM((CH,), jnp.int32),
        pltpu.VMEM((CH, H), jnp.float32),
        pltpu.VMEM((CH, H), jnp.float32),
        pltpu.VMEM((256,), jnp.float32),
        pltpu.VMEM((CH,), jnp.float32),
        pltpu.VMEM((H,), jnp.float32),
        pltpu.SemaphoreType.DMA,
        pltpu.SemaphoreType.DMA,
    ],
)(_scorer_body)


NPS = 25000
ACC_R = 25088
RPT = ACC_R // NS
KA = 512
EPT = E_PAD // NS
NCHA = EPT // KA
H2 = H // 2


def _agg_body(xs_hbm, d0_hbm, d1_hbm, src_hbm, out_hbm,
              idxs, idxd, rows, zbuf, accsh, semg, semsc):
    c = lax.axis_index("c")
    s = lax.axis_index("s")

    def zrow(r, carry):
        for k in range(H2 // 16):
            zbuf[r, pl.ds(k * 16, 16)] = jnp.zeros((16,), jnp.float32)
        return carry

    lax.fori_loop(0, 224, zrow, 0)
    for j in range(RPT // 224):
        pltpu.sync_copy(zbuf, accsh.at[pl.ds(s * RPT + j * 224, 224)])
    plsc.subcore_barrier()

    base_t = s * EPT

    def chunk(t, carry):
        base = base_t + t * KA
        pltpu.sync_copy(src_hbm.at[pl.ds(base, KA)], idxs)

        @pl.when(c == 0)
        def _():
            pltpu.sync_copy(d0_hbm.at[pl.ds(base, KA)], idxd)

        @pl.when(c == 1)
        def _():
            pltpu.sync_copy(d1_hbm.at[pl.ds(base, KA)], idxd)

        pltpu.async_copy(xs_hbm.at[idxs], rows, semg).wait()
        pltpu.async_copy(rows, accsh.at[idxd], semsc, add=True).wait()
        return carry

    lax.fori_loop(0, NCHA, chunk, 0)
    plsc.subcore_barrier()

    row0 = s * RPT

    @pl.when(s < NS - 1)
    def _():
        pltpu.sync_copy(accsh.at[pl.ds(row0, RPT)],
                        out_hbm.at[pl.ds(c * NPS + row0, RPT)])

    @pl.when(s == NS - 1)
    def _():
        last = NPS - (NS - 1) * RPT
        pltpu.sync_copy(accsh.at[pl.ds(row0, last)],
                        out_hbm.at[pl.ds(c * NPS + row0, last)])


_agg = functools.partial(
    pl.kernel,
    out_type=jax.ShapeDtypeStruct((N, H2), jnp.float32),
    mesh=_sc_mesh,
    compiler_params=pltpu.CompilerParams(
        needs_layout_passes=False, use_tc_tiling_on_sc=False
    ),
    scratch_types=[
        pltpu.VMEM((KA,), jnp.int32),
        pltpu.VMEM((KA,), jnp.int32),
        pltpu.VMEM((KA, H2), jnp.float32),
        pltpu.VMEM((224, H2), jnp.float32),
        pltpu.VMEM_SHARED((ACC_R, H2), jnp.float32),
        pltpu.SemaphoreType.DMA,
        pltpu.SemaphoreType.DMA,
    ],
)(_agg_body)


EDGE_BLK = 8192


def kernel(x, edge_index, W1, b1, W2, b2, Wm1, bm1, Wm2, bm2):
    src = edge_index[0].astype(jnp.int32)
    dst = edge_index[1].astype(jnp.int32)

    deg = jnp.ones((N,), jnp.float32).at[dst].add(1.0)
    dinv = lax.rsqrt(deg)

    src_pad = jnp.pad(src, (0, E_PAD - E))
    dump = jnp.full((E_PAD - E,), NPS, jnp.int32)
    d0 = jnp.concatenate([jnp.where(dst < NPS, dst, NPS), dump])
    d1 = jnp.concatenate(
        [jnp.where(dst >= NPS, dst - NPS, NPS), dump])

    def layer(h, W, b):
        xs = dinv[:, None] * (h @ W)
        acc0 = _agg(xs[:, :H2], d0, d1, src_pad)
        acc1 = _agg(xs[:, H2:], d0, d1, src_pad)
        acc = jnp.concatenate([acc0, acc1], axis=1)
        return jax.nn.relu(dinv[:, None] * (acc + xs) + b)

    h = layer(x, W1, b1)
    h = layer(h, W2, b2)

    A = h @ Wm1[:H] + bm1
    B = h @ Wm1[H:]

    col_pad = jnp.pad(dst, (0, E_PAD - E))
    out = _scorer(A, B, src_pad, col_pad, Wm2[:, 0])
    return out[:E] + bm2[0]


# speedup vs baseline: 5.8372x; 1.0675x over previous
import functools

import jax
import jax.numpy as jnp
from jax import lax
from jax.experimental import pallas as pl
from jax.experimental.pallas import tpu as pltpu
from jax.experimental.pallas import tpu_sc as plsc

N = 50000
E = 800000
H = 64

NC = 2
NS = 16
NW = NC * NS

EPW = 25600
E_PAD = NW * EPW
CH = 256
NCHUNK = EPW // CH

_sc_mesh = plsc.VectorSubcoreMesh(core_axis_name="c", subcore_axis_name="s")


def _scorer_body(a_hbm, b_hbm, row_hbm, col_hbm, w_hbm, out_hbm,
                 idxa0, idxa1, idxb0, idxb1, rowsa0, rowsa1, rowsb0, rowsb1,
                 tbuf, outv0, outv1, wv, sema0, sema1, semb0, semb1):
    c = lax.axis_index("c")
    s = lax.axis_index("s")
    wid = s * NC + c
    base0 = wid * EPW
    idxa = (idxa0, idxa1)
    idxb = (idxb0, idxb1)
    rowsa = (rowsa0, rowsa1)
    rowsb = (rowsb0, rowsb1)
    outv = (outv0, outv1)
    sema = (sema0, sema1)
    semb = (semb0, semb1)
    pltpu.sync_copy(w_hbm, wv)
    iota16 = lax.iota(jnp.int32, 16)

    def stage(t, b):
        base = base0 + t * CH
        pltpu.sync_copy(row_hbm.at[pl.ds(base, CH)], idxa[b])
        pltpu.sync_copy(col_hbm.at[pl.ds(base, CH)], idxb[b])
        pltpu.async_copy(a_hbm.at[idxa[b]], rowsa[b], sema[b])
        pltpu.async_copy(b_hbm.at[idxb[b]], rowsb[b], semb[b])

    stage(0, 0)
    stage(1, 1)

    def pair(i, carry):
        t0 = i * 2
        for b in range(2):
            t = t0 + b
            base = base0 + t * CH
            pltpu.make_async_copy(a_hbm.at[idxa[b]], rowsa[b],
                                  sema[b]).wait()
            pltpu.make_async_copy(b_hbm.at[idxb[b]], rowsb[b],
                                  semb[b]).wait()
            wvs = [wv[pl.ds(k * 16, 16)] for k in range(4)]
            ra, rb, ov = rowsa[b], rowsb[b], outv[b]

            def group(gi, carry2):
                e0 = gi * 16
                for e in range(16):
                    zs = []
                    for k in range(4):
                        av = ra[e0 + e, pl.ds(k * 16, 16)]
                        bv = rb[e0 + e, pl.ds(k * 16, 16)]
                        zs.append(jnp.maximum(av + bv, 0.0) * wvs[k])
                    acc = (zs[0] + zs[1]) + (zs[2] + zs[3])
                    plsc.store_scatter(tbuf, [iota16 * 16 + e], acc)
                vals = [tbuf[pl.ds(l * 16, 16)] for l in range(16)]
                while len(vals) > 1:
                    vals = [vals[i2] + vals[i2 + 1]
                            for i2 in range(0, len(vals), 2)]
                ov[pl.ds(e0, 16)] = vals[0]
                return carry2

            lax.fori_loop(0, CH // 16, group, 0)
            pltpu.sync_copy(ov, out_hbm.at[pl.ds(base, CH)])

            @pl.when(t + 2 < NCHUNK)
            def _(b=b, t=t):
                stage(t + 2, b)

        return carry

    lax.fori_loop(0, NCHUNK // 2, pair, 0)


_scorer = functools.partial(
    pl.kernel,
    out_type=jax.ShapeDtypeStruct((E_PAD,), jnp.float32),
    mesh=_sc_mesh,
    compiler_params=pltpu.CompilerParams(
        needs_layout_passes=False, use_tc_tiling_on_sc=False
    ),
    scratch_types=[
        pltpu.VMEM((CH,), jnp.int32),
        pltpu.VMEM((CH,), jnp.int32),
        pltpu.VMEM((CH,), jnp.int32),
        pltpu.VMEM((CH,), jnp.int32),
        pltpu.VMEM((CH, H), jnp.float32),
        pltpu.VMEM((CH, H), jnp.float32),
        pltpu.VMEM((CH, H), jnp.float32),
        pltpu.VMEM((CH, H), jnp.float32),
        pltpu.VMEM((256,), jnp.float32),
        pltpu.VMEM((CH,), jnp.float32),
        pltpu.VMEM((CH,), jnp.float32),
        pltpu.VMEM((H,), jnp.float32),
        pltpu.SemaphoreType.DMA,
        pltpu.SemaphoreType.DMA,
        pltpu.SemaphoreType.DMA,
        pltpu.SemaphoreType.DMA,
    ],
)(_scorer_body)


NPS = 25000
ACC_R = 25088
RPT = ACC_R // NS
KA = 1024
EPT = E_PAD // NS
NCHA = EPT // KA
H2 = H // 2


def _agg_body(xs_hbm, d0_hbm, d1_hbm, src_hbm, out_hbm,
              idxs0, idxs1, idxd0, idxd1, rows0, rows1, zbuf, accsh,
              semg0, semg1, semsc0, semsc1):
    c = lax.axis_index("c")
    s = lax.axis_index("s")
    idxs = (idxs0, idxs1)
    idxd = (idxd0, idxd1)
    rows = (rows0, rows1)
    semg = (semg0, semg1)
    semsc = (semsc0, semsc1)

    def zrow(r, carry):
        for k in range(H2 // 16):
            zbuf[r, pl.ds(k * 16, 16)] = jnp.zeros((16,), jnp.float32)
        return carry

    lax.fori_loop(0, 224, zrow, 0)
    for j in range(RPT // 224):
        pltpu.sync_copy(zbuf, accsh.at[pl.ds(s * RPT + j * 224, 224)])
    plsc.subcore_barrier()

    base_t = s * EPT

    def stage(t, b):
        base = base_t + t * KA
        pltpu.sync_copy(src_hbm.at[pl.ds(base, KA)], idxs[b])

        @pl.when(c == 0)
        def _():
            pltpu.sync_copy(d0_hbm.at[pl.ds(base, KA)], idxd[b])

        @pl.when(c == 1)
        def _():
            pltpu.sync_copy(d1_hbm.at[pl.ds(base, KA)], idxd[b])

        pltpu.async_copy(xs_hbm.at[idxs[b]], rows[b], semg[b])

    stage(0, 0)
    stage(1, 1)

    def pair(i, carry):
        t0 = i * 2
        descs = []
        for b in range(2):
            pltpu.make_async_copy(xs_hbm.at[idxs[b]], rows[b], semg[b]).wait()
            descs.append(
                pltpu.async_copy(rows[b], accsh.at[idxd[b]], semsc[b],
                                 add=True))
        for b in range(2):
            descs[b].wait()

            @pl.when(t0 + b + 2 < NCHA)
            def _(b=b):
                stage(t0 + b + 2, b)

        return carry

    lax.fori_loop(0, NCHA // 2, pair, 0)
    plsc.subcore_barrier()

    row0 = s * RPT

    @pl.when(s < NS - 1)
    def _():
        pltpu.sync_copy(accsh.at[pl.ds(row0, RPT)],
                        out_hbm.at[pl.ds(c * NPS + row0, RPT)])

    @pl.when(s == NS - 1)
    def _():
        last = NPS - (NS - 1) * RPT
        pltpu.sync_copy(accsh.at[pl.ds(row0, last)],
                        out_hbm.at[pl.ds(c * NPS + row0, last)])


_agg = functools.partial(
    pl.kernel,
    out_type=jax.ShapeDtypeStruct((N, H2), jnp.float32),
    mesh=_sc_mesh,
    compiler_params=pltpu.CompilerParams(
        needs_layout_passes=False, use_tc_tiling_on_sc=False
    ),
    scratch_types=[
        pltpu.VMEM((KA,), jnp.int32),
        pltpu.VMEM((KA,), jnp.int32),
        pltpu.VMEM((KA,), jnp.int32),
        pltpu.VMEM((KA,), jnp.int32),
        pltpu.VMEM((KA, H2), jnp.float32),
        pltpu.VMEM((KA, H2), jnp.float32),
        pltpu.VMEM((224, H2), jnp.float32),
        pltpu.VMEM_SHARED((ACC_R, H2), jnp.float32),
        pltpu.SemaphoreType.DMA,
        pltpu.SemaphoreType.DMA,
        pltpu.SemaphoreType.DMA,
        pltpu.SemaphoreType.DMA,
    ],
)(_agg_body)


EDGE_BLK = 8192


def kernel(x, edge_index, W1, b1, W2, b2, Wm1, bm1, Wm2, bm2):
    src = edge_index[0].astype(jnp.int32)
    dst = edge_index[1].astype(jnp.int32)

    deg = jnp.ones((N,), jnp.float32).at[dst].add(1.0)
    dinv = lax.rsqrt(deg)

    src_pad = jnp.pad(src, (0, E_PAD - E))
    dump = jnp.full((E_PAD - E,), NPS, jnp.int32)
    d0 = jnp.concatenate([jnp.where(dst < NPS, dst, NPS), dump])
    d1 = jnp.concatenate(
        [jnp.where(dst >= NPS, dst - NPS, NPS), dump])

    def layer(h, W, b):
        xs = dinv[:, None] * (h @ W)
        acc0 = _agg(xs[:, :H2], d0, d1, src_pad)
        acc1 = _agg(xs[:, H2:], d0, d1, src_pad)
        acc = jnp.concatenate([acc0, acc1], axis=1)
        return jax.nn.relu(dinv[:, None] * (acc + xs) + b)

    h = layer(x, W1, b1)
    h = layer(h, W2, b2)

    A = h @ Wm1[:H] + bm1
    B = h @ Wm1[H:]

    col_pad = jnp.pad(dst, (0, E_PAD - E))
    out = _scorer(A, B, src_pad, col_pad, Wm2[:, 0])
    return out[:E] + bm2[0]


# speedup vs baseline: 8.9195x; 1.5281x over previous
import functools

import jax
import jax.numpy as jnp
from jax import lax
from jax.experimental import pallas as pl
from jax.experimental.pallas import tpu as pltpu
from jax.experimental.pallas import tpu_sc as plsc

N = 50000
E = 800000
H = 64

NC = 2
NS = 16
NW = NC * NS

EPW = 25600
E_PAD = NW * EPW
CH = 256
NCHUNK = EPW // CH

_sc_mesh = plsc.VectorSubcoreMesh(core_axis_name="c", subcore_axis_name="s")


def _scorer_body(a_hbm, b_hbm, row_hbm, col_hbm, w_hbm, out_hbm,
                 idxa0, idxa1, idxb0, idxb1, rowsa0, rowsa1, rowsb0, rowsb1,
                 tbuf, outv0, outv1, wv, sema0, sema1, semb0, semb1):
    c = lax.axis_index("c")
    s = lax.axis_index("s")
    wid = s * NC + c
    base0 = wid * EPW
    idxa = (idxa0, idxa1)
    idxb = (idxb0, idxb1)
    rowsa = (rowsa0, rowsa1)
    rowsb = (rowsb0, rowsb1)
    outv = (outv0, outv1)
    sema = (sema0, sema1)
    semb = (semb0, semb1)
    pltpu.sync_copy(w_hbm, wv)
    iota16 = lax.iota(jnp.int32, 16)

    def stage(t, b):
        base = base0 + t * CH
        pltpu.sync_copy(row_hbm.at[pl.ds(base, CH)], idxa[b])
        pltpu.sync_copy(col_hbm.at[pl.ds(base, CH)], idxb[b])
        pltpu.async_copy(a_hbm.at[idxa[b]], rowsa[b], sema[b])
        pltpu.async_copy(b_hbm.at[idxb[b]], rowsb[b], semb[b])

    stage(0, 0)
    stage(1, 1)

    def pair(i, carry):
        t0 = i * 2
        for b in range(2):
            t = t0 + b
            base = base0 + t * CH
            pltpu.make_async_copy(a_hbm.at[idxa[b]], rowsa[b],
                                  sema[b]).wait()
            pltpu.make_async_copy(b_hbm.at[idxb[b]], rowsb[b],
                                  semb[b]).wait()
            wvs = [wv[pl.ds(k * 16, 16)] for k in range(4)]
            ra, rb, ov = rowsa[b], rowsb[b], outv[b]

            def group(gi, carry2):
                e0 = gi * 16
                for e in range(16):
                    zs = []
                    for k in range(4):
                        av = ra[e0 + e, pl.ds(k * 16, 16)]
                        bv = rb[e0 + e, pl.ds(k * 16, 16)]
                        zs.append(jnp.maximum(av + bv, 0.0) * wvs[k])
                    acc = (zs[0] + zs[1]) + (zs[2] + zs[3])
                    plsc.store_scatter(tbuf, [iota16 * 16 + e], acc)
                vals = [tbuf[pl.ds(l * 16, 16)] for l in range(16)]
                while len(vals) > 1:
                    vals = [vals[i2] + vals[i2 + 1]
                            for i2 in range(0, len(vals), 2)]
                ov[pl.ds(e0, 16)] = vals[0]
                return carry2

            lax.fori_loop(0, CH // 16, group, 0)
            pltpu.sync_copy(ov, out_hbm.at[pl.ds(base, CH)])

            @pl.when(t + 2 < NCHUNK)
            def _(b=b, t=t):
                stage(t + 2, b)

        return carry

    lax.fori_loop(0, NCHUNK // 2, pair, 0)


_scorer = functools.partial(
    pl.kernel,
    out_type=jax.ShapeDtypeStruct((E_PAD,), jnp.float32),
    mesh=_sc_mesh,
    compiler_params=pltpu.CompilerParams(
        needs_layout_passes=False, use_tc_tiling_on_sc=False
    ),
    scratch_types=[
        pltpu.VMEM((CH,), jnp.int32),
        pltpu.VMEM((CH,), jnp.int32),
        pltpu.VMEM((CH,), jnp.int32),
        pltpu.VMEM((CH,), jnp.int32),
        pltpu.VMEM((CH, H), jnp.float32),
        pltpu.VMEM((CH, H), jnp.float32),
        pltpu.VMEM((CH, H), jnp.float32),
        pltpu.VMEM((CH, H), jnp.float32),
        pltpu.VMEM((256,), jnp.float32),
        pltpu.VMEM((CH,), jnp.float32),
        pltpu.VMEM((CH,), jnp.float32),
        pltpu.VMEM((H,), jnp.float32),
        pltpu.SemaphoreType.DMA,
        pltpu.SemaphoreType.DMA,
        pltpu.SemaphoreType.DMA,
        pltpu.SemaphoreType.DMA,
    ],
)(_scorer_body)


H2 = H // 2
ACC_R = 50176
RPT = ACC_R // NS
KA = 256
EPT = E_PAD // NS
NCHA = EPT // KA


def _agg_body(xs2_hbm, dsta_hbm, src_hbm, out0_hbm, out1_hbm,
              idxs0, idxs1, idxd0, idxd1, rows0, rows1, zbuf, accsh,
              semg0, semg1, semsc0, semsc1):
    c = lax.axis_index("c")
    s = lax.axis_index("s")
    idxs = (idxs0, idxs1)
    idxd = (idxd0, idxd1)
    rows = (rows0, rows1)
    semg = (semg0, semg1)
    semsc = (semsc0, semsc1)
    off = c * N

    def zrow(r, carry):
        for k in range(H2 // 16):
            zbuf[r, pl.ds(k * 16, 16)] = jnp.zeros((16,), jnp.float32)
        return carry

    lax.fori_loop(0, 224, zrow, 0)
    for j in range(RPT // 224):
        pltpu.sync_copy(zbuf, accsh.at[pl.ds(s * RPT + j * 224, 224)])
    plsc.subcore_barrier()

    base_t = s * EPT

    def stage(t, b):
        base = base_t + t * KA
        pltpu.sync_copy(src_hbm.at[pl.ds(base, KA)], idxs[b])
        pltpu.sync_copy(dsta_hbm.at[pl.ds(base, KA)], idxd[b])
        for i in range(KA // 16):
            idxs[b][pl.ds(i * 16, 16)] = idxs[b][pl.ds(i * 16, 16)] + off
        pltpu.async_copy(xs2_hbm.at[idxs[b]], rows[b], semg[b])

    stage(0, 0)
    stage(1, 1)

    def pair(i, carry):
        t0 = i * 2
        descs = []
        for b in range(2):
            pltpu.make_async_copy(xs2_hbm.at[idxs[b]], rows[b],
                                  semg[b]).wait()
            descs.append(
                pltpu.async_copy(rows[b], accsh.at[idxd[b]], semsc[b],
                                 add=True))
        for b in range(2):
            descs[b].wait()

            @pl.when(t0 + b + 2 < NCHA)
            def _(b=b):
                stage(t0 + b + 2, b)

        return carry

    lax.fori_loop(0, NCHA // 2, pair, 0)
    plsc.subcore_barrier()

    row0 = s * RPT
    last = N - (NS - 1) * RPT

    for ci, oref in ((0, out0_hbm), (1, out1_hbm)):

        @pl.when(c == ci)
        def _(oref=oref):

            @pl.when(s < NS - 1)
            def _():
                pltpu.sync_copy(accsh.at[pl.ds(row0, RPT)],
                                oref.at[pl.ds(row0, RPT)])

            @pl.when(s == NS - 1)
            def _():
                pltpu.sync_copy(accsh.at[pl.ds(row0, last)],
                                oref.at[pl.ds(row0, last)])


_agg = functools.partial(
    pl.kernel,
    out_type=(jax.ShapeDtypeStruct((N, H2), jnp.float32),
              jax.ShapeDtypeStruct((N, H2), jnp.float32)),
    mesh=_sc_mesh,
    compiler_params=pltpu.CompilerParams(
        needs_layout_passes=False, use_tc_tiling_on_sc=False
    ),
    scratch_types=[
        pltpu.VMEM((KA,), jnp.int32),
        pltpu.VMEM((KA,), jnp.int32),
        pltpu.VMEM((KA,), jnp.int32),
        pltpu.VMEM((KA,), jnp.int32),
        pltpu.VMEM((KA, H2), jnp.float32),
        pltpu.VMEM((KA, H2), jnp.float32),
        pltpu.VMEM((224, H2), jnp.float32),
        pltpu.VMEM_SHARED((ACC_R, H2), jnp.float32),
        pltpu.SemaphoreType.DMA,
        pltpu.SemaphoreType.DMA,
        pltpu.SemaphoreType.DMA,
        pltpu.SemaphoreType.DMA,
    ],
)(_agg_body)


EDGE_BLK = 8192


def kernel(x, edge_index, W1, b1, W2, b2, Wm1, bm1, Wm2, bm2):
    src = edge_index[0].astype(jnp.int32)
    dst = edge_index[1].astype(jnp.int32)

    deg = jnp.ones((N,), jnp.float32).at[dst].add(1.0)
    dinv = lax.rsqrt(deg)

    src_pad = jnp.pad(src, (0, E_PAD - E))
    dsta_pad = jnp.concatenate(
        [dst, jnp.full((E_PAD - E,), N, jnp.int32)])

    def layer(h, W, b):
        xs = dinv[:, None] * (h @ W)
        xs2 = jnp.concatenate([xs[:, :H2], xs[:, H2:]], axis=0)
        acc0, acc1 = _agg(xs2, dsta_pad, src_pad)
        acc = jnp.concatenate([acc0, acc1], axis=1)
        return jax.nn.relu(dinv[:, None] * (acc + xs) + b)

    h = layer(x, W1, b1)
    h = layer(h, W2, b2)

    A = h @ Wm1[:H] + bm1
    B = h @ Wm1[H:]

    col_pad = jnp.pad(dst, (0, E_PAD - E))
    out = _scorer(A, B, src_pad, col_pad, Wm2[:, 0])
    return out[:E] + bm2[0]
